# baseline (speedup 1.0000x reference)
import functools
import numpy as np
import jax
import jax.numpy as jnp
from jax import lax
from jax.experimental import pallas as pl
from jax.experimental.pallas import tpu as pltpu
from jax.experimental.pallas import tpu_sc as plsc

N = 10000
NPAD = 10240
E = 320000
D = 128
DH = 64
L = 8
ALPHA = 0.1
THETA = 0.5

NC = 2
NS = 16
EB = 125
ROWS_E = E // EB
ROWS_PER_TILE_AGG = ROWS_E // NS
ROWS_PER_TILE_DEG = ROWS_E // (NC * NS)
NODES_PER_TILE = NPAD // NS
ECH = 32


NBUF = 8


def _agg_body(curs_hbm, srcm_hbm, dstm_hbm, zeros_hbm, raw_hbm,
              acc, sidx, didx, *bufs):
    c = lax.axis_index("c")
    s = lax.axis_index("s")
    r0 = s * NODES_PER_TILE
    pltpu.sync_copy(zeros_hbm, acc.at[pl.ds(r0, NODES_PER_TILE)])
    eb0 = s * ROWS_PER_TILE_AGG
    plsc.subcore_barrier()

    tbl = curs_hbm.at[c]
    rowb = bufs[:NBUF]
    gsem = bufs[NBUF:2 * NBUF]
    ssem = bufs[2 * NBUF:]

    @pl.loop(0, ROWS_PER_TILE_AGG // ECH)
    def _(ch):
        pltpu.sync_copy(srcm_hbm.at[pl.ds(eb0 + ch * ECH, ECH)], sidx)
        pltpu.sync_copy(dstm_hbm.at[pl.ds(eb0 + ch * ECH, ECH)], didx)
        gd = [None] * ECH
        sd = [None] * ECH
        for j in range(ECH):
            b = j % NBUF
            if j >= NBUF:
                sd[j - NBUF].wait()
            gd[j] = pltpu.async_copy(tbl.at[sidx.at[j]], rowb[b], gsem[b])
            if j:
                pb = (j - 1) % NBUF
                gd[j - 1].wait()
                sd[j - 1] = pltpu.async_copy(
                    rowb[pb], acc.at[didx.at[j - 1]], ssem[pb], add=True)
        gd[ECH - 1].wait()
        lb = (ECH - 1) % NBUF
        sd[ECH - 1] = pltpu.async_copy(
            rowb[lb], acc.at[didx.at[ECH - 1]], ssem[lb], add=True)
        for j in range(ECH - NBUF, ECH):
            sd[j].wait()

    plsc.subcore_barrier()
    pltpu.sync_copy(acc.at[pl.ds(r0, NODES_PER_TILE)],
                    raw_hbm.at[c, pl.ds(r0, NODES_PER_TILE)])


@functools.cache
def _agg_call():
    return pl.kernel(
        _agg_body,
        out_type=jax.ShapeDtypeStruct((NC, NPAD, DH), jnp.float32),
        mesh=plsc.VectorSubcoreMesh(core_axis_name="c", subcore_axis_name="s"),
        compiler_params=pltpu.CompilerParams(
            use_tc_tiling_on_sc=False,
            disable_bounds_checks=True,
            disable_semaphore_checks=True,
        ),
        scratch_types=[
            pltpu.VMEM_SHARED((NPAD, DH), jnp.float32),
            pltpu.VMEM((ECH, EB), jnp.int32),
            pltpu.VMEM((ECH, EB), jnp.int32),
        ] + [pltpu.VMEM((EB, DH), jnp.float32) for _ in range(NBUF)]
          + [pltpu.SemaphoreType.DMA for _ in range(2 * NBUF)],
    )


ECH_DEG = 16


def _deg_body(dstm_hbm, ones_hbm, zeros_hbm, degp_hbm,
              degtab, didx, ones_v, sd0):
    c = lax.axis_index("c")
    s = lax.axis_index("s")
    r0 = s * NODES_PER_TILE
    pltpu.sync_copy(zeros_hbm, degtab.at[pl.ds(r0, NODES_PER_TILE)])
    pltpu.sync_copy(ones_hbm, ones_v)
    eb0 = (c * NS + s) * ROWS_PER_TILE_DEG
    plsc.subcore_barrier()

    @pl.loop(0, ROWS_PER_TILE_DEG // ECH_DEG)
    def _(ch):
        pltpu.sync_copy(dstm_hbm.at[pl.ds(eb0 + ch * ECH_DEG, ECH_DEG)], didx)
        ds = [pltpu.async_copy(ones_v, degtab.at[didx.at[j]], sd0,
                               add=True) for j in range(ECH_DEG)]
        for d in ds:
            d.wait()

    plsc.subcore_barrier()
    pltpu.sync_copy(degtab.at[pl.ds(r0, NODES_PER_TILE)],
                    degp_hbm.at[c, pl.ds(r0, NODES_PER_TILE)])


@functools.cache
def _deg_call():
    return pl.kernel(
        _deg_body,
        out_type=jax.ShapeDtypeStruct((NC, NPAD, DH), jnp.float32),
        mesh=plsc.VectorSubcoreMesh(core_axis_name="c", subcore_axis_name="s"),
        compiler_params=pltpu.CompilerParams(
            use_tc_tiling_on_sc=False,
            disable_bounds_checks=True,
            disable_semaphore_checks=True,
        ),
        scratch_types=[
            pltpu.VMEM_SHARED((NPAD, DH), jnp.float32),
            pltpu.VMEM((ECH_DEG, EB), jnp.int32),
            pltpu.VMEM((EB, DH), jnp.float32),
            pltpu.SemaphoreType.DMA,
        ],
    )


BN = 1024
GRID = NPAD // BN


def _row_spec():
    return pl.BlockSpec((BN, D), lambda j: (j, 0))


def _half_spec():
    return pl.BlockSpec((NC, BN, DH), lambda j: (0, j, 0))


def _dinv_spec():
    return pl.BlockSpec((BN, DH), lambda j: (j, 0))


def _full_spec(shape):
    nd = len(shape)
    return pl.BlockSpec(shape, lambda j: (0,) * nd)


def _prep_tc(degp_ref, x_ref, w0_ref, b0_ref, dinvb_ref, x0_ref, curs_ref):
    deg = degp_ref[0] + degp_ref[1]
    dh = jnp.where(deg > 0, lax.rsqrt(jnp.maximum(deg, 1e-12)), 0.0)
    h = jax.nn.relu(
        jnp.dot(x_ref[...], w0_ref[...], preferred_element_type=jnp.float32)
        + b0_ref[...])
    dinvb_ref[...] = dh
    x0_ref[0] = h[:, :DH]
    x0_ref[1] = h[:, DH:]
    curs_ref[0] = h[:, :DH] * dh
    curs_ref[1] = h[:, DH:] * dh


def _layer_tc(beta, raw_ref, x0_ref, dinvb_ref, wc_ref, out_ref):
    dinv = dinvb_ref[...]
    t_lo = (1.0 - ALPHA) * (dinv * raw_ref[0]) + ALPHA * x0_ref[0]
    t_hi = (1.0 - ALPHA) * (dinv * raw_ref[1]) + ALPHA * x0_ref[1]
    t = jnp.concatenate([t_lo, t_hi], axis=1)
    u = (1.0 - beta) * t + beta * jnp.dot(
        t, wc_ref[...], preferred_element_type=jnp.float32)
    cur = jax.nn.relu(u)
    out_ref[0] = cur[:, :DH] * dinv
    out_ref[1] = cur[:, DH:] * dinv


def _last_tc(beta, raw_ref, x0_ref, dinvb_ref, wc_ref, w1_ref, b1_ref,
             out_ref):
    dinv = dinvb_ref[...]
    t_lo = (1.0 - ALPHA) * (dinv * raw_ref[0]) + ALPHA * x0_ref[0]
    t_hi = (1.0 - ALPHA) * (dinv * raw_ref[1]) + ALPHA * x0_ref[1]
    t = jnp.concatenate([t_lo, t_hi], axis=1)
    u = (1.0 - beta) * t + beta * jnp.dot(
        t, wc_ref[...], preferred_element_type=jnp.float32)
    cur = jax.nn.relu(u)
    out_ref[...] = (
        jnp.dot(cur, w1_ref[...], preferred_element_type=jnp.float32)
        + b1_ref[...])


@jax.jit
def kernel(x, edge_index, lin0_W, lin0_b, convW, lin1_W, lin1_b):
    srcm = edge_index[0].reshape(ROWS_E, EB)
    dstm = edge_index[1].reshape(ROWS_E, EB)
    zeros64 = jnp.zeros((NODES_PER_TILE, DH), jnp.float32)
    ones64 = jnp.ones((EB, DH), jnp.float32)
    xp = jnp.zeros((NPAD, D), jnp.float32).at[:N].set(x)

    degp = _deg_call()(dstm, ones64, zeros64)

    prep = pl.pallas_call(
        _prep_tc,
        grid=(GRID,),
        in_specs=[
            _half_spec(),
            _row_spec(),
            _full_spec((D, D)),
            _full_spec((1, D)),
        ],
        out_specs=[_dinv_spec(), _half_spec(), _half_spec()],
        out_shape=[
            jax.ShapeDtypeStruct((NPAD, DH), jnp.float32),
            jax.ShapeDtypeStruct((NC, NPAD, DH), jnp.float32),
            jax.ShapeDtypeStruct((NC, NPAD, DH), jnp.float32),
        ],
    )
    dinvb, x0h, curs = prep(degp, xp, lin0_W, lin0_b.reshape(1, D))

    for i in range(L):
        beta = float(np.log(THETA / (i + 1) + 1.0))
        raw = _agg_call()(curs, srcm, dstm, zeros64)
        if i < L - 1:
            layer = pl.pallas_call(
                functools.partial(_layer_tc, beta),
                grid=(GRID,),
                in_specs=[_half_spec(), _half_spec(), _dinv_spec(),
                          _full_spec((D, D))],
                out_specs=_half_spec(),
                out_shape=jax.ShapeDtypeStruct((NC, NPAD, DH), jnp.float32),
            )
            curs = layer(raw, x0h, dinvb, convW[i])
        else:
            last = pl.pallas_call(
                functools.partial(_last_tc, beta),
                grid=(GRID,),
                in_specs=[_half_spec(), _half_spec(), _dinv_spec(),
                          _full_spec((D, D)), _full_spec((D, D)),
                          _full_spec((1, D))],
                out_specs=_row_spec(),
                out_shape=jax.ShapeDtypeStruct((NPAD, D), jnp.float32),
            )
            out = last(raw, x0h, dinvb, convW[i], lin1_W,
                       lin1_b.reshape(1, D))
    return out[:N]


# speedup vs baseline: 1.0005x; 1.0005x over previous
import functools
import numpy as np
import jax
import jax.numpy as jnp
from jax import lax
from jax.experimental import pallas as pl
from jax.experimental.pallas import tpu as pltpu
from jax.experimental.pallas import tpu_sc as plsc

N = 10000
NPAD = 10240
E = 320000
D = 128
DH = 64
L = 8
ALPHA = 0.1
THETA = 0.5

NC = 2
NS = 16
EB = 125
ROWS_E = E // EB
ROWS_PER_TILE_AGG = ROWS_E // NS
ROWS_PER_TILE_DEG = ROWS_E // (NC * NS)
NODES_PER_TILE = NPAD // NS
ECH = 32


NBUF = 8


def _agg_body(curs_hbm, srcm_hbm, dstm_hbm, zeros_hbm, raw_hbm,
              acc, sidx, didx, *bufs):
    c = lax.axis_index("c")
    s = lax.axis_index("s")
    r0 = s * NODES_PER_TILE
    pltpu.sync_copy(zeros_hbm, acc.at[pl.ds(r0, NODES_PER_TILE)])
    eb0 = s * ROWS_PER_TILE_AGG
    plsc.subcore_barrier()

    tbl = curs_hbm.at[c]
    rowb = bufs[:NBUF]
    gsem = bufs[NBUF:2 * NBUF]
    ssem = bufs[2 * NBUF:]

    @pl.loop(0, ROWS_PER_TILE_AGG // ECH)
    def _(ch):
        pltpu.sync_copy(srcm_hbm.at[pl.ds(eb0 + ch * ECH, ECH)], sidx)
        pltpu.sync_copy(dstm_hbm.at[pl.ds(eb0 + ch * ECH, ECH)], didx)
        gd = [None] * ECH
        sd = [None] * ECH
        for j in range(ECH):
            b = j % NBUF
            if j >= NBUF:
                sd[j - NBUF].wait()
            gd[j] = pltpu.async_copy(tbl.at[sidx.at[j]], rowb[b], gsem[b])
            if j:
                pb = (j - 1) % NBUF
                gd[j - 1].wait()
                sd[j - 1] = pltpu.async_copy(
                    rowb[pb], acc.at[didx.at[j - 1]], ssem[pb], add=True)
        gd[ECH - 1].wait()
        lb = (ECH - 1) % NBUF
        sd[ECH - 1] = pltpu.async_copy(
            rowb[lb], acc.at[didx.at[ECH - 1]], ssem[lb], add=True)
        for j in range(ECH - NBUF, ECH):
            sd[j].wait()

    plsc.subcore_barrier()
    pltpu.sync_copy(acc.at[pl.ds(r0, NODES_PER_TILE)],
                    raw_hbm.at[c, pl.ds(r0, NODES_PER_TILE)])


@functools.cache
def _agg_call():
    return pl.kernel(
        _agg_body,
        out_type=jax.ShapeDtypeStruct((NC, NPAD, DH), jnp.float32),
        mesh=plsc.VectorSubcoreMesh(core_axis_name="c", subcore_axis_name="s"),
        compiler_params=pltpu.CompilerParams(use_tc_tiling_on_sc=False),
        scratch_types=[
            pltpu.VMEM_SHARED((NPAD, DH), jnp.float32),
            pltpu.VMEM((ECH, EB), jnp.int32),
            pltpu.VMEM((ECH, EB), jnp.int32),
        ] + [pltpu.VMEM((EB, DH), jnp.float32) for _ in range(NBUF)]
          + [pltpu.SemaphoreType.DMA for _ in range(2 * NBUF)],
    )


ECH_DEG = 16


def _deg_body(dstm_hbm, ones_hbm, zeros_hbm, degp_hbm,
              degtab, didx, ones_v, sd0):
    c = lax.axis_index("c")
    s = lax.axis_index("s")
    r0 = s * NODES_PER_TILE
    pltpu.sync_copy(zeros_hbm, degtab.at[pl.ds(r0, NODES_PER_TILE)])
    pltpu.sync_copy(ones_hbm, ones_v)
    eb0 = (c * NS + s) * ROWS_PER_TILE_DEG
    plsc.subcore_barrier()

    @pl.loop(0, ROWS_PER_TILE_DEG // ECH_DEG)
    def _(ch):
        pltpu.sync_copy(dstm_hbm.at[pl.ds(eb0 + ch * ECH_DEG, ECH_DEG)], didx)
        ds = [pltpu.async_copy(ones_v, degtab.at[didx.at[j]], sd0,
                               add=True) for j in range(ECH_DEG)]
        for d in ds:
            d.wait()

    plsc.subcore_barrier()
    pltpu.sync_copy(degtab.at[pl.ds(r0, NODES_PER_TILE)],
                    degp_hbm.at[c, pl.ds(r0, NODES_PER_TILE)])


@functools.cache
def _deg_call():
    return pl.kernel(
        _deg_body,
        out_type=jax.ShapeDtypeStruct((NC, NPAD, DH), jnp.float32),
        mesh=plsc.VectorSubcoreMesh(core_axis_name="c", subcore_axis_name="s"),
        compiler_params=pltpu.CompilerParams(use_tc_tiling_on_sc=False),
        scratch_types=[
            pltpu.VMEM_SHARED((NPAD, DH), jnp.float32),
            pltpu.VMEM((ECH_DEG, EB), jnp.int32),
            pltpu.VMEM((EB, DH), jnp.float32),
            pltpu.SemaphoreType.DMA,
        ],
    )


BN = 1024
GRID = NPAD // BN


def _row_spec():
    return pl.BlockSpec((BN, D), lambda j: (j, 0))


def _half_spec():
    return pl.BlockSpec((NC, BN, DH), lambda j: (0, j, 0))


def _dinv_spec():
    return pl.BlockSpec((BN, DH), lambda j: (j, 0))


def _full_spec(shape):
    nd = len(shape)
    return pl.BlockSpec(shape, lambda j: (0,) * nd)


def _prep_tc(degp_ref, x_ref, w0_ref, b0_ref, dinvb_ref, x0_ref, curs_ref):
    deg = degp_ref[0] + degp_ref[1]
    dh = jnp.where(deg > 0, lax.rsqrt(jnp.maximum(deg, 1e-12)), 0.0)
    h = jax.nn.relu(
        jnp.dot(x_ref[...], w0_ref[...], preferred_element_type=jnp.float32)
        + b0_ref[...])
    dinvb_ref[...] = dh
    x0_ref[0] = h[:, :DH]
    x0_ref[1] = h[:, DH:]
    curs_ref[0] = h[:, :DH] * dh
    curs_ref[1] = h[:, DH:] * dh


def _layer_tc(beta, raw_ref, x0_ref, dinvb_ref, wc_ref, out_ref):
    dinv = dinvb_ref[...]
    t_lo = (1.0 - ALPHA) * (dinv * raw_ref[0]) + ALPHA * x0_ref[0]
    t_hi = (1.0 - ALPHA) * (dinv * raw_ref[1]) + ALPHA * x0_ref[1]
    t = jnp.concatenate([t_lo, t_hi], axis=1)
    u = (1.0 - beta) * t + beta * jnp.dot(
        t, wc_ref[...], preferred_element_type=jnp.float32)
    cur = jax.nn.relu(u)
    out_ref[0] = cur[:, :DH] * dinv
    out_ref[1] = cur[:, DH:] * dinv


def _last_tc(beta, raw_ref, x0_ref, dinvb_ref, wc_ref, w1_ref, b1_ref,
             out_ref):
    dinv = dinvb_ref[...]
    t_lo = (1.0 - ALPHA) * (dinv * raw_ref[0]) + ALPHA * x0_ref[0]
    t_hi = (1.0 - ALPHA) * (dinv * raw_ref[1]) + ALPHA * x0_ref[1]
    t = jnp.concatenate([t_lo, t_hi], axis=1)
    u = (1.0 - beta) * t + beta * jnp.dot(
        t, wc_ref[...], preferred_element_type=jnp.float32)
    cur = jax.nn.relu(u)
    out_ref[...] = (
        jnp.dot(cur, w1_ref[...], preferred_element_type=jnp.float32)
        + b1_ref[...])


@jax.jit
def kernel(x, edge_index, lin0_W, lin0_b, convW, lin1_W, lin1_b):
    srcm = edge_index[0].reshape(ROWS_E, EB)
    dstm = edge_index[1].reshape(ROWS_E, EB)
    zeros64 = jnp.zeros((NODES_PER_TILE, DH), jnp.float32)
    ones64 = jnp.ones((EB, DH), jnp.float32)
    xp = jnp.zeros((NPAD, D), jnp.float32).at[:N].set(x)

    degp = _deg_call()(dstm, ones64, zeros64)

    prep = pl.pallas_call(
        _prep_tc,
        grid=(GRID,),
        in_specs=[
            _half_spec(),
            _row_spec(),
            _full_spec((D, D)),
            _full_spec((1, D)),
        ],
        out_specs=[_dinv_spec(), _half_spec(), _half_spec()],
        out_shape=[
            jax.ShapeDtypeStruct((NPAD, DH), jnp.float32),
            jax.ShapeDtypeStruct((NC, NPAD, DH), jnp.float32),
            jax.ShapeDtypeStruct((NC, NPAD, DH), jnp.float32),
        ],
    )
    dinvb, x0h, curs = prep(degp, xp, lin0_W, lin0_b.reshape(1, D))

    for i in range(L):
        beta = float(np.log(THETA / (i + 1) + 1.0))
        raw = _agg_call()(curs, srcm, dstm, zeros64)
        if i < L - 1:
            layer = pl.pallas_call(
                functools.partial(_layer_tc, beta),
                grid=(GRID,),
                in_specs=[_half_spec(), _half_spec(), _dinv_spec(),
                          _full_spec((D, D))],
                out_specs=_half_spec(),
                out_shape=jax.ShapeDtypeStruct((NC, NPAD, DH), jnp.float32),
            )
            curs = layer(raw, x0h, dinvb, convW[i])
        else:
            last = pl.pallas_call(
                functools.partial(_last_tc, beta),
                grid=(GRID,),
                in_specs=[_half_spec(), _half_spec(), _dinv_spec(),
                          _full_spec((D, D)), _full_spec((D, D)),
                          _full_spec((1, D))],
                out_specs=_row_spec(),
                out_shape=jax.ShapeDtypeStruct((NPAD, D), jnp.float32),
            )
            out = last(raw, x0h, dinvb, convW[i], lin1_W,
                       lin1_b.reshape(1, D))
    return out[:N]
